# baseline (speedup 1.0000x reference)
import functools

import jax
import jax.numpy as jnp
from jax import lax
from jax.experimental import pallas as pl
from jax.experimental.pallas import tpu as pltpu
from jax.experimental.pallas import tpu_sc as plsc

_NUM_ENTITIES = 100000
_DIM = 64
_BATCH = 16384
_MARGIN = 1.0

_NC = 2
_NS = 16
_NW = _NC * _NS
_PER_W = _BATCH // _NW
_CHUNK = 128
_NCHUNK = _PER_W // _CHUNK
_GROUPS = _CHUNK // 16


def _rsqrt_nr(x):
    i = plsc.bitcast(x, jnp.int32)
    i = jnp.int32(0x5F3759DF) - lax.shift_right_arithmetic(i, jnp.int32(1))
    y = plsc.bitcast(i, jnp.float32)
    half_x = 0.5 * x
    for _ in range(3):
        y = y * (1.5 - half_x * y * y)
    return y


def _lane_sum(x):
    lanes = jnp.arange(16, dtype=jnp.int32)
    for k in (1, 2, 4, 8):
        perm = lanes ^ k
        x = x + x.at[perm].get(mode="promise_in_bounds")
    return x


def _scores_for_group(h_ref, r_ref, t_ref, g):
    lanes = jnp.arange(16, dtype=jnp.int32)
    hrt_v = jnp.zeros((16,), jnp.float32)
    nh_v = jnp.zeros((16,), jnp.float32)
    nt_v = jnp.zeros((16,), jnp.float32)
    for j in range(16):
        row = g * 16 + j
        hs = [h_ref[row, pl.ds(k * 16, 16)] for k in range(_DIM // 16)]
        rs = [r_ref[row, pl.ds(k * 16, 16)] for k in range(_DIM // 16)]
        ts = [t_ref[row, pl.ds(k * 16, 16)] for k in range(_DIM // 16)]
        prod = [hs[k] * rs[k] * ts[k] for k in range(_DIM // 16)]
        hh = [hs[k] * hs[k] for k in range(_DIM // 16)]
        tt = [ts[k] * ts[k] for k in range(_DIM // 16)]
        s_hrt = (prod[0] + prod[1]) + (prod[2] + prod[3])
        s_nh = (hh[0] + hh[1]) + (hh[2] + hh[3])
        s_nt = (tt[0] + tt[1]) + (tt[2] + tt[3])
        is_j = lanes == j
        hrt_v = jnp.where(is_j, _lane_sum(s_hrt), hrt_v)
        nh_v = jnp.where(is_j, _lane_sum(s_nh), nh_v)
        nt_v = jnp.where(is_j, _lane_sum(s_nt), nt_v)
    x = jnp.maximum(nh_v * nt_v, jnp.float32(1e-30))
    return hrt_v * _rsqrt_nr(x)


def _sc_kernel(ph_hbm, pr_hbm, pt_hbm, nh_hbm, nr_hbm, nt_hbm,
               ent_hbm, rel_hbm, out_hbm,
               ph_i, pr_i, pt_i, nh_i, nr_i, nt_i,
               ph_r, pr_r, pt_r, nh_r, nr_r, nt_r,
               acc_v, sem):
    wid = lax.axis_index("s") * _NC + lax.axis_index("c")

    def chunk_body(c, acc):
        off = wid * _PER_W + c * _CHUNK
        sl = pl.ds(off, _CHUNK)
        pltpu.sync_copy(ph_hbm.at[sl], ph_i)
        pltpu.sync_copy(pr_hbm.at[sl], pr_i)
        pltpu.sync_copy(pt_hbm.at[sl], pt_i)
        pltpu.sync_copy(nh_hbm.at[sl], nh_i)
        pltpu.sync_copy(nr_hbm.at[sl], nr_i)
        pltpu.sync_copy(nt_hbm.at[sl], nt_i)
        cps = [
            pltpu.async_copy(ent_hbm.at[ph_i], ph_r, sem),
            pltpu.async_copy(rel_hbm.at[pr_i], pr_r, sem),
            pltpu.async_copy(ent_hbm.at[pt_i], pt_r, sem),
            pltpu.async_copy(ent_hbm.at[nh_i], nh_r, sem),
            pltpu.async_copy(rel_hbm.at[nr_i], nr_r, sem),
            pltpu.async_copy(ent_hbm.at[nt_i], nt_r, sem),
        ]
        for cp in cps:
            cp.wait()

        def group_body(g, a):
            p = _scores_for_group(ph_r, pr_r, pt_r, g)
            n = _scores_for_group(nh_r, nr_r, nt_r, g)
            return a + jnp.maximum(jnp.float32(0.0), _MARGIN - p + n)

        return lax.fori_loop(0, _GROUPS, group_body, acc)

    acc = lax.fori_loop(0, _NCHUNK, chunk_body, jnp.zeros((16,), jnp.float32))
    acc_v[...] = acc
    pltpu.sync_copy(acc_v, out_hbm.at[wid])


def kernel(pos_exmpls, neg_exmpls, entity_emb, relation_emb):
    mesh = plsc.VectorSubcoreMesh(core_axis_name="c", subcore_axis_name="s")
    f = pl.kernel(
        _sc_kernel,
        mesh=mesh,
        compiler_params=pltpu.CompilerParams(
            needs_layout_passes=False, use_tc_tiling_on_sc=False
        ),
        out_type=jax.ShapeDtypeStruct((_NW, 16), jnp.float32),
        scratch_types=[
            pltpu.VMEM((_CHUNK,), jnp.int32) for _ in range(6)
        ] + [
            pltpu.VMEM((_CHUNK, _DIM), jnp.float32) for _ in range(6)
        ] + [
            pltpu.VMEM((16,), jnp.float32),
            pltpu.SemaphoreType.DMA,
        ],
    )
    partials = f(
        pos_exmpls[:, 0], pos_exmpls[:, 1], pos_exmpls[:, 2],
        neg_exmpls[:, 0], neg_exmpls[:, 1], neg_exmpls[:, 2],
        entity_emb, relation_emb,
    )
    return jnp.sum(partials) * jnp.float32(1.0 / _BATCH)
